# baseline (speedup 1.0000x reference)
import functools

import jax
import jax.numpy as jnp
from jax import lax
from jax.experimental import pallas as pl
from jax.experimental.pallas import tpu as pltpu
from jax.experimental.pallas import tpu_sc as plsc

N = 5000
M = 20000
N_PAD = 5120
M_PAD = 20480
R = 512
C = 4096
RS = 128
K = 8

NUM_WORKERS = 32
B_PER_W = N_PAD // NUM_WORKERS
LANES = 16


def _nn_idx_body(p_ref, t_ref, p2_ref, t2_ref, idx_ref, rmin_ref, rchk_ref):
    j = pl.program_id(1)

    @pl.when(j == 0)
    def _():
        rmin_ref[...] = jnp.full((R, 128), jnp.inf, jnp.float32)
        rchk_ref[...] = jnp.zeros((R, 128), jnp.float32)

    dotr = lax.dot_general(p_ref[...], t_ref[...], (((1,), (1,)), ((), ())),
                           preferred_element_type=jnp.float32)
    t2 = t2_ref[...]
    p2 = p2_ref[...]

    for sub in range(R // RS):
        rows = pl.ds(sub * RS, RS)
        rm = rmin_ref[rows, :]
        rc = rchk_ref[rows, :]
        p2s = p2[sub * RS:(sub + 1) * RS, :]
        for k in range(C // 128):
            cols = slice(k * 128, (k + 1) * 128)
            ch = (p2s + t2[:, cols]) - dotr[sub * RS:(sub + 1) * RS, cols]
            ck = jnp.float32(j * (C // 128) + k)
            cond = ch < rm
            rm = jnp.where(cond, ch, rm)
            rc = jnp.where(cond, ck, rc)
        rmin_ref[rows, :] = rm
        rchk_ref[rows, :] = rc

    @pl.when(j == pl.num_programs(1) - 1)
    def _():
        rmin_f = rmin_ref[...]
        lanef = lax.broadcasted_iota(jnp.int32, (R, 128), 1).astype(jnp.float32)
        col = rchk_ref[...] * 128.0 + lanef
        bmin = jnp.min(rmin_f, axis=1, keepdims=True)
        bidx = jnp.min(jnp.where(rmin_f == bmin, col, jnp.float32(2 ** 30)),
                       axis=1, keepdims=True)
        idx_ref[...] = bidx.astype(jnp.int32)


def _nn_indices(p_pad, t_pad, p2_pad, t2_pad):
    call = pl.pallas_call(
        _nn_idx_body,
        grid=(N_PAD // R, M_PAD // C),
        in_specs=[
            pl.BlockSpec((R, 4), lambda i, j: (i, 0)),
            pl.BlockSpec((C, 4), lambda i, j: (j, 0)),
            pl.BlockSpec((R, 1), lambda i, j: (i, 0)),
            pl.BlockSpec((1, C), lambda i, j: (0, j)),
        ],
        out_specs=pl.BlockSpec((R, 1), lambda i, j: (i, 0)),
        out_shape=jax.ShapeDtypeStruct((N_PAD, 1), jnp.int32),
        scratch_shapes=[
            pltpu.VMEM((R, 128), jnp.float32),
            pltpu.VMEM((R, 128), jnp.float32),
        ],
        compiler_params=pltpu.CompilerParams(
            dimension_semantics=("parallel", "arbitrary"),
        ),
    )
    return call(p_pad, t_pad, p2_pad, t2_pad)


def _sc_mse_body(idx_hbm, pint_hbm, tint_hbm, out_hbm,
                 idx_v, pint_v, tint_v, acc_v):
    wid = lax.axis_index("s") * 2 + lax.axis_index("c")
    base = wid * B_PER_W
    pltpu.sync_copy(idx_hbm.at[pl.ds(base, B_PER_W)], idx_v)
    pltpu.sync_copy(pint_hbm.at[pl.ds(base, B_PER_W)], pint_v)
    pltpu.sync_copy(tint_hbm, tint_v)

    lane = lax.iota(jnp.int32, LANES)
    acc = jnp.zeros((LANES,), jnp.float32)
    for i in range(B_PER_W // LANES):
        iv = idx_v[pl.ds(i * LANES, LANES)]
        g = plsc.load_gather(tint_v, [iv])
        pv = pint_v[pl.ds(i * LANES, LANES)]
        row = base + i * LANES + lane
        d = jnp.where(row < N, pv - g, 0.0)
        acc = acc + d * d
    acc_v[...] = acc
    pltpu.sync_copy(acc_v, out_hbm.at[wid])


@functools.lru_cache(maxsize=1)
def _sc_mse():
    return pl.kernel(
        _sc_mse_body,
        out_type=jax.ShapeDtypeStruct((NUM_WORKERS, LANES), jnp.float32),
        mesh=plsc.VectorSubcoreMesh(core_axis_name="c", subcore_axis_name="s"),
        compiler_params=pltpu.CompilerParams(needs_layout_passes=False),
        scratch_types=[
            pltpu.VMEM((B_PER_W,), jnp.int32),
            pltpu.VMEM((B_PER_W,), jnp.float32),
            pltpu.VMEM((M,), jnp.float32),
            pltpu.VMEM((LANES,), jnp.float32),
        ],
    )


@jax.jit
def kernel(pred, target):
    p = pred[:, :3]
    t = target[:, :3]
    p2 = jnp.sum(p * p, axis=1, keepdims=True)
    t2 = jnp.sum(t * t, axis=1, keepdims=True).T
    scale = jnp.array([[2.0, 2.0, 2.0, 0.0]], jnp.float32)
    p_pad = jnp.zeros((N_PAD, 4), jnp.float32).at[:N].set(pred * scale)
    t_pad = jnp.zeros((M_PAD, 4), jnp.float32).at[:M].set(target)
    p2_pad = jnp.zeros((N_PAD, 1), jnp.float32).at[:N].set(p2)
    t2_pad = jnp.full((1, M_PAD), 1e30, jnp.float32).at[:, :M].set(t2)
    idx = _nn_indices(p_pad, t_pad, p2_pad, t2_pad)[:, 0]

    pint_pad = jnp.zeros((N_PAD,), jnp.float32).at[:N].set(pred[:, 3])
    partials = _sc_mse()(idx, pint_pad, target[:, 3])
    return jnp.sum(partials) / N


# speedup vs baseline: 1.1410x; 1.1410x over previous
import functools

import jax
import jax.numpy as jnp
from jax import lax
from jax.experimental import pallas as pl
from jax.experimental.pallas import tpu as pltpu
from jax.experimental.pallas import tpu_sc as plsc

N = 5000
M = 20000
N_PAD = 5120
M_PAD = 20480
R = 512
C = 4096
RS = 128
K = 8

NUM_WORKERS = 32
B_PER_W = N_PAD // NUM_WORKERS
LANES = 16


def _nn_idx_body(p_ref, tT_ref, p2_ref, idx_ref, rmin_ref, rchk_ref):
    j = pl.program_id(1)

    @pl.when(j == 0)
    def _():
        rmin_ref[...] = jnp.full((R, 128), jnp.inf, jnp.float32)
        rchk_ref[...] = jnp.zeros((R, 128), jnp.float32)

    p = p_ref[...]
    tT = tT_ref[...]
    t2 = jnp.sum(tT * tT, axis=0, keepdims=True)
    colr = j * C + lax.broadcasted_iota(jnp.int32, (1, C), 1)
    t2 = jnp.where(colr < M, t2, jnp.float32(1e30))
    dotr = jnp.dot(p, tT, preferred_element_type=jnp.float32)
    p2 = p2_ref[...]

    for sub in range(R // RS):
        rows = pl.ds(sub * RS, RS)
        rm = rmin_ref[rows, :]
        rc = rchk_ref[rows, :]
        p2s = p2[sub * RS:(sub + 1) * RS, :]
        for k in range(C // 128):
            cols = slice(k * 128, (k + 1) * 128)
            ch = (p2s + t2[:, cols]) - dotr[sub * RS:(sub + 1) * RS, cols]
            ck = jnp.float32(j * (C // 128) + k)
            cond = ch < rm
            rm = jnp.where(cond, ch, rm)
            rc = jnp.where(cond, ck, rc)
        rmin_ref[rows, :] = rm
        rchk_ref[rows, :] = rc

    @pl.when(j == pl.num_programs(1) - 1)
    def _():
        rmin_f = rmin_ref[...]
        lanef = lax.broadcasted_iota(jnp.int32, (R, 128), 1).astype(jnp.float32)
        col = rchk_ref[...] * 128.0 + lanef
        bmin = jnp.min(rmin_f, axis=1, keepdims=True)
        bidx = jnp.min(jnp.where(rmin_f == bmin, col, jnp.float32(2 ** 30)),
                       axis=1, keepdims=True)
        idx_ref[...] = bidx.astype(jnp.int32)


def _nn_indices(p_pad, tT_pad, p2_pad):
    call = pl.pallas_call(
        _nn_idx_body,
        grid=(N_PAD // R, M_PAD // C),
        in_specs=[
            pl.BlockSpec((R, K), lambda i, j: (i, 0)),
            pl.BlockSpec((K, C), lambda i, j: (0, j)),
            pl.BlockSpec((R, 1), lambda i, j: (i, 0)),
        ],
        out_specs=pl.BlockSpec((R, 1), lambda i, j: (i, 0)),
        out_shape=jax.ShapeDtypeStruct((N_PAD, 1), jnp.int32),
        scratch_shapes=[
            pltpu.VMEM((R, 128), jnp.float32),
            pltpu.VMEM((R, 128), jnp.float32),
        ],
        compiler_params=pltpu.CompilerParams(
            dimension_semantics=("parallel", "arbitrary"),
        ),
    )
    return call(p_pad, tT_pad, p2_pad)


def _sc_mse_body(idx_hbm, pint_hbm, tint_hbm, out_hbm,
                 idx_v, pint_v, tint_v, acc_v):
    wid = lax.axis_index("s") * 2 + lax.axis_index("c")
    base = wid * B_PER_W
    pltpu.sync_copy(idx_hbm.at[pl.ds(base, B_PER_W)], idx_v)
    pltpu.sync_copy(pint_hbm.at[pl.ds(base, B_PER_W)], pint_v)
    pltpu.sync_copy(tint_hbm, tint_v)

    lane = lax.iota(jnp.int32, LANES)
    acc = jnp.zeros((LANES,), jnp.float32)
    for i in range(B_PER_W // LANES):
        iv = idx_v[pl.ds(i * LANES, LANES)]
        g = plsc.load_gather(tint_v, [iv])
        pv = pint_v[pl.ds(i * LANES, LANES)]
        row = base + i * LANES + lane
        d = jnp.where(row < N, pv - g, 0.0)
        acc = acc + d * d
    acc_v[...] = acc
    pltpu.sync_copy(acc_v, out_hbm.at[wid])


@functools.lru_cache(maxsize=1)
def _sc_mse():
    return pl.kernel(
        _sc_mse_body,
        out_type=jax.ShapeDtypeStruct((NUM_WORKERS, LANES), jnp.float32),
        mesh=plsc.VectorSubcoreMesh(core_axis_name="c", subcore_axis_name="s"),
        compiler_params=pltpu.CompilerParams(needs_layout_passes=False),
        scratch_types=[
            pltpu.VMEM((B_PER_W,), jnp.int32),
            pltpu.VMEM((B_PER_W,), jnp.float32),
            pltpu.VMEM((M,), jnp.float32),
            pltpu.VMEM((LANES,), jnp.float32),
        ],
    )


@jax.jit
def kernel(pred, target):
    p = pred[:, :3]
    p2 = jnp.sum(p * p, axis=1, keepdims=True)
    p_pad = jnp.zeros((N_PAD, K), jnp.float32).at[:N, :3].set(2.0 * p)
    tT_pad = jnp.zeros((K, M_PAD), jnp.float32).at[:3, :M].set(target[:, :3].T)
    p2_pad = jnp.zeros((N_PAD, 1), jnp.float32).at[:N].set(p2)
    idx = _nn_indices(p_pad, tT_pad, p2_pad)[:, 0]

    pint_pad = jnp.zeros((N_PAD,), jnp.float32).at[:N].set(pred[:, 3])
    partials = _sc_mse()(idx, pint_pad, target[:, 3])
    return jnp.sum(partials) / N
